# baseline (speedup 1.0000x reference)
import functools

import jax
import jax.numpy as jnp
from jax import lax
from jax.experimental import pallas as pl
from jax.experimental.pallas import tpu as pltpu
from jax.experimental.pallas import tpu_sc as plsc

ROWS = 8192
DIM = 1024
NUM_CORES = 2
ROWS_PER_CORE = ROWS // NUM_CORES
NBUF = 4
CHUNK = 256
NCHUNK = ROWS_PER_CORE // CHUNK
SLACK = 2

_mesh = plsc.VectorSubcoreMesh(core_axis_name="c", subcore_axis_name="s")


@functools.partial(
    pl.kernel,
    mesh=_mesh,
    out_type=jax.ShapeDtypeStruct((ROWS, DIM), jnp.float32),
    scratch_types=(
        [pltpu.VMEM_SHARED((CHUNK, DIM), jnp.float32)] * NBUF
        + [pltpu.SemaphoreType.DMA] * (2 * NBUF)
    ),
)
def _sc_copy(src_hbm, out_hbm, *scratch):
    bufs = scratch[:NBUF]
    in_sems = scratch[NBUF:2 * NBUF]
    out_sems = scratch[2 * NBUF:]

    cid = lax.axis_index("c")
    sid = lax.axis_index("s")
    base = cid * ROWS_PER_CORE

    def cp_in(g, b):
        return pltpu.make_async_copy(
            src_hbm.at[pl.ds(base + g * CHUNK, CHUNK)], bufs[b], in_sems[b])

    def cp_out(g, b):
        return pltpu.make_async_copy(
            bufs[b], out_hbm.at[pl.ds(base + g * CHUNK, CHUNK)], out_sems[b])

    @pl.when(sid == 0)
    def _():
        for b in range(min(NBUF, NCHUNK)):
            cp_in(b, b).start()
        waited_out = set()
        for g in range(NCHUNK):
            nxt = g + NBUF - SLACK
            if NBUF <= nxt < NCHUNK:
                prev = nxt - NBUF
                cp_out(prev, prev % NBUF).wait()
                waited_out.add(prev)
                cp_in(nxt, nxt % NBUF).start()
            cp_in(g, g % NBUF).wait()
            cp_out(g, g % NBUF).start()
        for g in range(NCHUNK):
            if g not in waited_out:
                cp_out(g, g % NBUF).wait()


def kernel(x, emb_weight):
    sl = x.shape[1]
    out = _sc_copy(emb_weight)
    return out[:sl]


# speedup vs baseline: 1.0361x; 1.0361x over previous
import functools

import jax
import jax.numpy as jnp
from jax import lax
from jax.experimental import pallas as pl
from jax.experimental.pallas import tpu as pltpu
from jax.experimental.pallas import tpu_sc as plsc

ROWS = 8192
DIM = 1024
NUM_CORES = 2
NUM_SUBCORES = 16
NUM_WORKERS = NUM_CORES * NUM_SUBCORES

T_ROWS = 4096
T_RPW = T_ROWS // NUM_WORKERS
T_NBUF = 4
T_CHUNK = 16
T_NCHUNK = T_RPW // T_CHUNK

S_ROWS = ROWS - T_ROWS
S_RPC = S_ROWS // NUM_CORES
S_NBUF = 4
S_CHUNK = 256
S_NCHUNK = S_RPC // S_CHUNK

_mesh = plsc.VectorSubcoreMesh(core_axis_name="c", subcore_axis_name="s")


@functools.partial(
    pl.kernel,
    mesh=_mesh,
    out_type=jax.ShapeDtypeStruct((ROWS, DIM), jnp.float32),
    scratch_types=(
        [pltpu.VMEM((T_CHUNK, DIM), jnp.float32)] * T_NBUF
        + [pltpu.VMEM_SHARED((S_CHUNK, DIM), jnp.float32)] * S_NBUF
        + [pltpu.SemaphoreType.DMA] * (2 * T_NBUF + 2 * S_NBUF)
    ),
)
def _sc_copy(src_hbm, out_hbm, *scratch):
    tbufs = scratch[:T_NBUF]
    sbufs = scratch[T_NBUF:T_NBUF + S_NBUF]
    sems = scratch[T_NBUF + S_NBUF:]
    t_in = sems[:T_NBUF]
    t_out = sems[T_NBUF:2 * T_NBUF]
    s_in = sems[2 * T_NBUF:2 * T_NBUF + S_NBUF]
    s_out = sems[2 * T_NBUF + S_NBUF:]

    cid = lax.axis_index("c")
    sid = lax.axis_index("s")
    wid = sid * NUM_CORES + cid
    t_base = wid * T_RPW
    s_base = T_ROWS + cid * S_RPC

    def t_cp_in(g, b):
        return pltpu.make_async_copy(
            src_hbm.at[pl.ds(t_base + g * T_CHUNK, T_CHUNK)], tbufs[b], t_in[b])

    def t_cp_out(g, b):
        return pltpu.make_async_copy(
            tbufs[b], out_hbm.at[pl.ds(t_base + g * T_CHUNK, T_CHUNK)], t_out[b])

    def s_cp_in(g, b):
        return pltpu.make_async_copy(
            src_hbm.at[pl.ds(s_base + g * S_CHUNK, S_CHUNK)], sbufs[b], s_in[b])

    def s_cp_out(g, b):
        return pltpu.make_async_copy(
            sbufs[b], out_hbm.at[pl.ds(s_base + g * S_CHUNK, S_CHUNK)], s_out[b])

    for b in range(min(T_NBUF, T_NCHUNK)):
        t_cp_in(b, b).start()

    @pl.when(sid == 0)
    def _():
        for b in range(min(S_NBUF, S_NCHUNK)):
            s_cp_in(b, b).start()

    t_waited = set()
    s_waited = set()
    n_iter = max(T_NCHUNK, S_NCHUNK)
    for g in range(n_iter):
        if g < T_NCHUNK:
            nxt = g + T_NBUF - 1
            if T_NBUF <= nxt < T_NCHUNK:
                prev = nxt - T_NBUF
                t_cp_out(prev, prev % T_NBUF).wait()
                t_waited.add(prev)
                t_cp_in(nxt, nxt % T_NBUF).start()
            t_cp_in(g, g % T_NBUF).wait()
            t_cp_out(g, g % T_NBUF).start()
        if g < S_NCHUNK:
            @pl.when(sid == 0)
            def _(g=g):
                nxt = g + S_NBUF - 1
                if S_NBUF <= nxt < S_NCHUNK:
                    prev = nxt - S_NBUF
                    s_cp_out(prev, prev % S_NBUF).wait()
                    s_cp_in(nxt, nxt % S_NBUF).start()
                s_cp_in(g, g % S_NBUF).wait()
                s_cp_out(g, g % S_NBUF).start()
            nxt = g + S_NBUF - 1
            if S_NBUF <= nxt < S_NCHUNK:
                s_waited.add(nxt - S_NBUF)

    for g in range(T_NCHUNK):
        if g not in t_waited:
            t_cp_out(g, g % T_NBUF).wait()

    @pl.when(sid == 0)
    def _():
        for g in range(S_NCHUNK):
            if g not in s_waited:
                s_cp_out(g, g % S_NBUF).wait()


def kernel(x, emb_weight):
    sl = x.shape[1]
    out = _sc_copy(emb_weight)
    return out[:sl]
